# Initial kernel scaffold; baseline (speedup 1.0000x reference)
import functools

import jax
import jax.numpy as jnp
from jax import lax
from jax.experimental import pallas as pl
from jax.experimental.pallas import tpu as pltpu
from jax.experimental.pallas import tpu_sc as plsc

_B = 16384
_NNUM = 13
_NCAT = 26
_D = 64
_ROWS = _NNUM + _NCAT
_L = 16
_NC = 2
_NS = 16
_NW = _NC * _NS
_PB = _B // _NW
_CB = 16
_NCH = _PB // _CB

_mesh = plsc.VectorSubcoreMesh(
    core_axis_name="c", subcore_axis_name="s", num_cores=_NC, num_subcores=_NS
)


@functools.partial(
    pl.kernel,
    out_type=jax.ShapeDtypeStruct((_B, _ROWS, _D), jnp.float32),
    mesh=_mesh,
    scratch_types=[
        pltpu.VMEM((_PB, _NCAT), jnp.int32),
        pltpu.VMEM((_PB, _NNUM), jnp.float32),
        pltpu.VMEM((_NNUM, _D), jnp.float32),
        pltpu.VMEM((_NNUM, _D), jnp.float32),
        pltpu.VMEM((_CB, _ROWS, _D), jnp.float32),
        pltpu.SemaphoreType.DMA,
        pltpu.SemaphoreType.DMA,
    ],
)
def _tokenizer_sc(table_hbm, idx_hbm, x_hbm, w_hbm, bias_hbm, out_hbm,
                  idx_v, x_v, w_v, b_v, buf, gsem, ssem):
    wid = lax.axis_index("s") * _NC + lax.axis_index("c")
    base = wid * _PB

    pltpu.async_copy(idx_hbm.at[pl.ds(base, _PB)], idx_v, ssem)
    pltpu.async_copy(x_hbm.at[pl.ds(base, _PB)], x_v, ssem)
    pltpu.async_copy(w_hbm, w_v, ssem)
    pltpu.async_copy(bias_hbm, b_v, ssem).wait()
    pltpu.make_async_copy(idx_hbm.at[pl.ds(base, _PB)], idx_v, ssem).wait()
    pltpu.make_async_copy(x_hbm.at[pl.ds(base, _PB)], x_v, ssem).wait()
    pltpu.make_async_copy(w_hbm, w_v, ssem).wait()

    @pl.loop(0, _NCH)
    def _chunk(s):
        row0 = s * _CB
        gathers = []
        for b in range(_CB):
            gathers.append(
                pltpu.async_copy(
                    table_hbm.at[idx_v.at[row0 + b]],
                    buf.at[b, pl.ds(_NNUM, _NCAT)],
                    gsem,
                )
            )
        for j in range(_D // _L):
            dj = pl.ds(j * _L, _L)
            for i in range(_NNUM):
                wj = w_v[i, dj]
                bj = b_v[i, dj]
                for b in range(_CB):
                    xs = x_v[row0 + b, i]
                    buf[b, i, dj] = jnp.full((_L,), xs, jnp.float32) * wj + bj
        for g in gathers:
            g.wait()
        pltpu.async_copy(buf, out_hbm.at[pl.ds(base + row0, _CB)], ssem).wait()


def kernel(x_num, x_cat, W_num, b_num, cat_table):
    idx = x_cat.astype(jnp.int32)
    x_flat = x_num.reshape(_B, _NNUM)
    return _tokenizer_sc(cat_table, idx, x_flat, W_num, b_num)


# speedup vs baseline: 3.2916x; 3.2916x over previous
import functools

import jax
import jax.numpy as jnp
from jax import lax
from jax.experimental import pallas as pl
from jax.experimental.pallas import tpu as pltpu
from jax.experimental.pallas import tpu_sc as plsc

_B = 16384
_NNUM = 13
_NCAT = 26
_D = 64
_ROWS = _NNUM + _NCAT
_L = 16
_NC = 2
_NS = 16
_NW = _NC * _NS
_PB = _B // _NW
_CB = 16
_NCH = _PB // _CB

_mesh = plsc.VectorSubcoreMesh(
    core_axis_name="c", subcore_axis_name="s", num_cores=_NC, num_subcores=_NS
)


@functools.partial(
    pl.kernel,
    out_type=jax.ShapeDtypeStruct((_B, _ROWS, _D), jnp.float32),
    mesh=_mesh,
    scratch_types=[
        pltpu.VMEM((_PB, _NCAT), jnp.int32),
        pltpu.VMEM((_PB, _L), jnp.float32),
        pltpu.VMEM((_NNUM, _D), jnp.float32),
        pltpu.VMEM((_NNUM, _D), jnp.float32),
        pltpu.VMEM((_CB, _ROWS, _D), jnp.float32),
        pltpu.SemaphoreType.DMA,
        pltpu.SemaphoreType.DMA,
    ],
    compiler_params=pltpu.CompilerParams(use_tc_tiling_on_sc=False),
)
def _tokenizer_sc(table_hbm, idx_hbm, x_hbm, w_hbm, bias_hbm, out_hbm,
                  idx_v, x_v, w_v, b_v, buf, gsem, ssem):
    wid = lax.axis_index("s") * _NC + lax.axis_index("c")
    base = wid * _PB

    pltpu.async_copy(idx_hbm.at[pl.ds(base, _PB)], idx_v, ssem)
    pltpu.async_copy(x_hbm.at[pl.ds(base, _PB)], x_v, ssem)
    pltpu.async_copy(w_hbm, w_v, ssem)
    pltpu.async_copy(bias_hbm, b_v, ssem).wait()
    pltpu.make_async_copy(idx_hbm.at[pl.ds(base, _PB)], idx_v, ssem).wait()
    pltpu.make_async_copy(x_hbm.at[pl.ds(base, _PB)], x_v, ssem).wait()
    pltpu.make_async_copy(w_hbm, w_v, ssem).wait()

    @pl.loop(0, _NCH)
    def _chunk(s):
        row0 = s * _CB
        gathers = []
        for b in range(_CB):
            gathers.append(
                pltpu.async_copy(
                    table_hbm.at[idx_v.at[row0 + b]],
                    buf.at[b, pl.ds(_NNUM, _NCAT)],
                    gsem,
                )
            )
        for b in range(_CB):
            xrow = x_v[row0 + b]
            for i in range(_NNUM):
                bc = jnp.full((_L,), xrow[i], jnp.float32)
                for j in range(_D // _L):
                    dj = pl.ds(j * _L, _L)
                    buf[b, i, dj] = bc * w_v[i, dj] + b_v[i, dj]
        for g in gathers:
            g.wait()
        pltpu.async_copy(buf, out_hbm.at[pl.ds(base + row0, _CB)], ssem).wait()


def kernel(x_num, x_cat, W_num, b_num, cat_table):
    idx = x_cat.astype(jnp.int32)
    x_pad = jnp.pad(x_num.reshape(_B, _NNUM), ((0, 0), (0, _L - _NNUM)))
    return _tokenizer_sc(cat_table, idx, x_pad, W_num, b_num)
